# baseline (speedup 1.0000x reference)
import functools
import math

import jax
import jax.numpy as jnp
from jax import lax
from jax.experimental import pallas as pl
from jax.experimental.pallas import tpu as pltpu
from jax.experimental.pallas import tpu_sc as plsc

_DEPTH = 128
_LANES = 16


@functools.cache
def _build(B, S, D, V):
    info = plsc.get_sparse_core_info()
    NC, NS, L = info.num_cores, info.num_subcores, info.num_lanes
    NW = NC * NS
    R = B * S
    rows_per_w = R // NW
    chunks = rows_per_w // S
    half = S // 2
    scale = math.sqrt(float(D))

    mesh = plsc.VectorSubcoreMesh(core_axis_name="c", subcore_axis_name="s")

    @functools.partial(
        pl.kernel,
        out_type=jax.ShapeDtypeStruct((R, D), jnp.float32),
        mesh=mesh,
        scratch_types=[
            pltpu.VMEM((2, half), jnp.int32),
            pltpu.VMEM((S, D), jnp.float32),
            pltpu.VMEM((S, D), jnp.float32),
            pltpu.SemaphoreType.DMA,
        ],
    )
    def emb_kernel(table_hbm, x2_hbm, enc_hbm, out_hbm, idx_v, rows_v, enc_v, sem):
        wid = lax.axis_index("s") * NC + lax.axis_index("c")
        pltpu.sync_copy(enc_hbm, enc_v)

        def chunk_body(ci, carry):
            row0 = wid * rows_per_w + ci * S
            pltpu.sync_copy(x2_hbm.at[pl.ds(wid * (2 * chunks) + ci * 2, 2)], idx_v)
            cp0 = pltpu.async_copy(
                table_hbm.at[idx_v.at[0]], rows_v.at[pl.ds(0, half)], sem)
            cp1 = pltpu.async_copy(
                table_hbm.at[idx_v.at[1]], rows_v.at[pl.ds(half, half)], sem)
            cp0.wait()
            cp1.wait()

            def row_body(r, c2):
                for c in range(D // L):
                    sl = pl.ds(c * L, L)
                    rows_v[r, sl] = rows_v[r, sl] * scale + enc_v[r, sl]
                return c2

            lax.fori_loop(0, S, row_body, 0, unroll=2)
            pltpu.sync_copy(rows_v, out_hbm.at[pl.ds(row0, S)])
            return carry

        lax.fori_loop(0, chunks, chunk_body, 0)

    return emb_kernel


def kernel(x, table, encoding):
    B, S = x.shape
    V, D = table.shape
    x2 = x.reshape(-1, S // 2).astype(jnp.int32)
    enc = encoding[:S, :]
    out = _build(B, S, D, V)(table, x2, enc)
    return out.reshape(B, S, D)
